# Initial kernel scaffold; baseline (speedup 1.0000x reference)
import jax
import jax.numpy as jnp
from jax.experimental import pallas as pl


def kernel(x, edge_index, edge_attr, weight, W_ih, W_hh, b_ih, b_hh, fc_w, fc_b, prob):
    raise NotImplementedError("write your pallas kernel here")


# speedup vs baseline: 3.6389x; 3.6389x over previous
import functools
import jax
import jax.numpy as jnp
from jax import lax
from jax.experimental import pallas as pl
from jax.experimental.pallas import tpu as pltpu
from jax.experimental.pallas import tpu_sc as plsc

N = 10000
E = 320000
D = 128
NC = 2
NS = 16
NW = NC * NS
EW = E // NW
CB = 80
NCH = EW // CB
S624 = 624
CHW = 104
NCHW = 6
TAIL = N - NS * S624
TSTART = NS * S624


def _sc_body(m_hbm, src_hbm, dst_hbm, attr_hbm, zero_hbm, out_hbm,
             srcv, dstv, attrv, rows, bounce, acc, sem):
  c = lax.axis_index("c")
  s = lax.axis_index("s")
  wid = c * NS + s

  start = s * S624
  pltpu.sync_copy(zero_hbm, bounce)
  for i in range(NCHW):
    pltpu.sync_copy(bounce, acc.at[pl.ds(start + i * CHW, CHW)])

  @pl.when(s == NS - 1)
  def _zero_tail():
    pltpu.sync_copy(bounce.at[pl.ds(0, TAIL)], acc.at[pl.ds(TSTART, TAIL)])
  plsc.subcore_barrier()

  def chunk(j, carry):
    base = wid * EW + j * CB
    pltpu.sync_copy(src_hbm.at[pl.ds(base, CB)], srcv)
    pltpu.sync_copy(dst_hbm.at[pl.ds(base, CB)], dstv)
    pltpu.sync_copy(attr_hbm.at[pl.ds(base, CB)], attrv)
    pltpu.async_copy(m_hbm.at[srcv], rows, sem).wait()

    def edge(e, carry2):
      ab = plsc.load_gather(attrv, [jnp.full((16,), e, jnp.int32)])
      for q in range(D // 16):
        rows[e, pl.ds(q * 16, 16)] = rows[e, pl.ds(q * 16, 16)] * ab
      return carry2
    lax.fori_loop(0, CB, edge, 0, unroll=2)

    pltpu.sync_copy(rows, acc.at[dstv], add=True)
    return carry
  lax.fori_loop(0, NCH, chunk, 0)

  plsc.subcore_barrier()
  for i in range(NCHW):
    pltpu.sync_copy(acc.at[pl.ds(start + i * CHW, CHW)], bounce)
    pltpu.sync_copy(bounce, out_hbm.at[c, pl.ds(start + i * CHW, CHW)])

  @pl.when(s == NS - 1)
  def _write_tail():
    pltpu.sync_copy(acc.at[pl.ds(TSTART, TAIL)], bounce.at[pl.ds(0, TAIL)])
    pltpu.sync_copy(bounce.at[pl.ds(0, TAIL)], out_hbm.at[c, pl.ds(TSTART, TAIL)])


def _sc_scatter(m, src3, dst3, attr, zeros):
  mesh = plsc.VectorSubcoreMesh(core_axis_name="c", subcore_axis_name="s")
  k = pl.kernel(
      _sc_body,
      out_type=jax.ShapeDtypeStruct((NC, N, D), jnp.float32),
      mesh=mesh,
      scratch_types=[
          pltpu.VMEM((CB,), jnp.int32),
          pltpu.VMEM((CB,), jnp.int32),
          pltpu.VMEM((CB,), jnp.float32),
          pltpu.VMEM((CB, D), jnp.float32),
          pltpu.VMEM((CHW, D), jnp.float32),
          pltpu.VMEM_SHARED((N, D), jnp.float32),
          pltpu.SemaphoreType.DMA,
      ],
      compiler_params=pltpu.CompilerParams(needs_layout_passes=False),
  )
  return k(m, src3, dst3, attr, zeros)


BN = 2000


def _mm_body(h_ref, w_ref, o_ref):
  o_ref[...] = jnp.dot(h_ref[...], w_ref[...],
                       preferred_element_type=jnp.float32)


def _matmul(h, w):
  return pl.pallas_call(
      _mm_body,
      grid=(N // BN,),
      in_specs=[
          pl.BlockSpec((BN, D), lambda i: (i, 0)),
          pl.BlockSpec((D, D), lambda i: (0, 0)),
      ],
      out_specs=pl.BlockSpec((BN, D), lambda i: (i, 0)),
      out_shape=jax.ShapeDtypeStruct((N, D), jnp.float32),
  )(h, w)


def _gru_math(p_ref, h_ref, wih_ref, whh_ref, bih_ref, bhh_ref):
  agg = p_ref[0] + p_ref[1]
  h = h_ref[...]
  gi = lax.dot_general(agg, wih_ref[...], (((1,), (1,)), ((), ())),
                       preferred_element_type=jnp.float32) + bih_ref[...]
  gh = lax.dot_general(h, whh_ref[...], (((1,), (1,)), ((), ())),
                       preferred_element_type=jnp.float32) + bhh_ref[...]
  r = jax.nn.sigmoid(gi[:, :D] + gh[:, :D])
  z = jax.nn.sigmoid(gi[:, D:2 * D] + gh[:, D:2 * D])
  n = jnp.tanh(gi[:, 2 * D:] + r * gh[:, 2 * D:])
  return (1.0 - z) * n + z * h


def _gru_mid_body(p_ref, h_ref, wih_ref, whh_ref, bih_ref, bhh_ref, wn_ref,
                  hn_ref, mn_ref):
  hn = _gru_math(p_ref, h_ref, wih_ref, whh_ref, bih_ref, bhh_ref)
  hn_ref[...] = hn
  mn_ref[...] = jnp.dot(hn, wn_ref[...], preferred_element_type=jnp.float32)


def _gru_last_body(p_ref, h_ref, wih_ref, whh_ref, bih_ref, bhh_ref,
                   fcw_ref, fcb_ref, out_ref):
  hn = _gru_math(p_ref, h_ref, wih_ref, whh_ref, bih_ref, bhh_ref)
  hr = jnp.maximum(hn, 0.0)
  out_ref[...] = jnp.sum(hr * fcw_ref[...], axis=1, keepdims=True) + fcb_ref[...]


_W_SPECS = [
    pl.BlockSpec((NC, BN, D), lambda i: (0, i, 0)),
    pl.BlockSpec((BN, D), lambda i: (i, 0)),
    pl.BlockSpec((3 * D, D), lambda i: (0, 0)),
    pl.BlockSpec((3 * D, D), lambda i: (0, 0)),
    pl.BlockSpec((1, 3 * D), lambda i: (0, 0)),
    pl.BlockSpec((1, 3 * D), lambda i: (0, 0)),
]


def _gru_mid(p, h, wih, whh, bih, bhh, wn):
  return pl.pallas_call(
      _gru_mid_body,
      grid=(N // BN,),
      in_specs=_W_SPECS + [pl.BlockSpec((D, D), lambda i: (0, 0))],
      out_specs=[
          pl.BlockSpec((BN, D), lambda i: (i, 0)),
          pl.BlockSpec((BN, D), lambda i: (i, 0)),
      ],
      out_shape=[
          jax.ShapeDtypeStruct((N, D), jnp.float32),
          jax.ShapeDtypeStruct((N, D), jnp.float32),
      ],
  )(p, h, wih, whh, bih, bhh, wn)


def _gru_last(p, h, wih, whh, bih, bhh, fcw, fcb):
  return pl.pallas_call(
      _gru_last_body,
      grid=(N // BN,),
      in_specs=_W_SPECS + [
          pl.BlockSpec((1, D), lambda i: (0, 0)),
          pl.BlockSpec((1, 1), lambda i: (0, 0)),
      ],
      out_specs=pl.BlockSpec((BN, 1), lambda i: (i, 0)),
      out_shape=jax.ShapeDtypeStruct((N, 1), jnp.float32),
  )(p, h, wih, whh, bih, bhh, fcw, fcb)


def kernel(x, edge_index, edge_attr, weight, W_ih, W_hh, b_ih, b_hh,
           fc_w, fc_b, prob=0):
  src3 = edge_index[0]
  dst3 = edge_index[1]
  zeros = jnp.zeros((CHW, D), jnp.float32)
  bih = b_ih.reshape(1, 3 * D)
  bhh = b_hh.reshape(1, 3 * D)
  fcb = fc_b.reshape(1, 1)

  m = _matmul(x, weight[0])
  p = _sc_scatter(m, src3, dst3, edge_attr, zeros)
  h, m = _gru_mid(p, x, W_ih, W_hh, bih, bhh, weight[1])
  p = _sc_scatter(m, src3, dst3, edge_attr, zeros)
  out = _gru_last(p, h, W_ih, W_hh, bih, bhh, fc_w, fcb)
  return out


# speedup vs baseline: 10.7844x; 2.9636x over previous
import functools
import jax
import jax.numpy as jnp
from jax import lax
from jax.experimental import pallas as pl
from jax.experimental.pallas import tpu as pltpu
from jax.experimental.pallas import tpu_sc as plsc

N = 10000
E = 320000
D = 128
NC = 2
NS = 16
NW = NC * NS
EW = E // NW
CB = 40
SUP = 50
NSUP = EW // (SUP * CB)
NBUF = 5
GDEP = 4
S624 = 624
CHW = 48
NCHW = 13
TAIL = N - NS * S624
TSTART = NS * S624


def _sc_body(m_hbm, src_hbm, dst_hbm, attr_hbm, zero_hbm, out_hbm,
             srcv, dstidx, attrv, rows, bounce, acc, gsem, ssem, dsem):
  c = lax.axis_index("c")
  s = lax.axis_index("s")
  wid = c * NS + s

  start = s * S624
  pltpu.sync_copy(zero_hbm, bounce)
  for i in range(NCHW):
    pltpu.sync_copy(bounce, acc.at[pl.ds(start + i * CHW, CHW)])

  @pl.when(s == NS - 1)
  def _zero_tail():
    pltpu.sync_copy(bounce.at[pl.ds(0, TAIL)], acc.at[pl.ds(TSTART, TAIL)])
  plsc.subcore_barrier()

  def start_gather(sj, g, b):
    pltpu.async_copy(m_hbm.at[srcv.at[pl.ds(g * CB, CB)]], rows.at[b],
                     gsem.at[b])
    pltpu.async_copy(dst_hbm.at[pl.ds(wid * EW + sj * SUP * CB + g * CB, CB)],
                     dstidx.at[b], dsem.at[b])

  def wait_dst(b):
    pltpu.make_async_copy(dst_hbm.at[pl.ds(0, CB)], dstidx.at[b],
                          dsem.at[b]).wait()

  def wait_gather(b):
    pltpu.make_async_copy(m_hbm.at[pl.ds(0, CB)], rows.at[b],
                          gsem.at[b]).wait()

  def wait_scatter(b):
    pltpu.make_async_copy(m_hbm.at[pl.ds(0, CB)], rows.at[b],
                          ssem.at[b]).wait()

  def super_chunk(sj, carry):
    pltpu.sync_copy(src_hbm.at[pl.ds(wid * EW + sj * SUP * CB, SUP * CB)],
                    srcv)
    pltpu.sync_copy(attr_hbm.at[pl.ds(wid * EW + sj * SUP * CB, SUP * CB)],
                    attrv)
    for b in range(GDEP):
      start_gather(sj, b, b)

    def block(kk, carry2):
      for b in range(NBUF):
        jj = kk * NBUF + b

        wait_gather(b)
        wait_dst(b)

        def edge(e, carry3):
          ab = plsc.load_gather(
              attrv, [jnp.full((16,), jj * CB + e, jnp.int32)])
          for q in range(D // 16):
            rows[b, e, pl.ds(q * 16, 16)] = (
                rows[b, e, pl.ds(q * 16, 16)] * ab)
          return carry3
        lax.fori_loop(0, CB, edge, 0, unroll=2)

        pltpu.async_copy(rows.at[b], acc.at[dstidx.at[b]], ssem.at[b],
                         add=True)

        g = jj + GDEP
        bn = (b + GDEP) % NBUF

        @pl.when((g < SUP) & (jj >= 1))
        def _():
          wait_scatter(bn)

        @pl.when(g < SUP)
        def _():
          start_gather(sj, g, bn)
      return carry2
    lax.fori_loop(0, SUP // NBUF, block, 0)

    for b in range(NBUF):
      wait_scatter(b)
    return carry
  lax.fori_loop(0, NSUP, super_chunk, 0)

  plsc.subcore_barrier()
  for i in range(NCHW):
    pltpu.sync_copy(acc.at[pl.ds(start + i * CHW, CHW)], bounce)
    pltpu.sync_copy(bounce, out_hbm.at[c, pl.ds(start + i * CHW, CHW)])

  @pl.when(s == NS - 1)
  def _write_tail():
    pltpu.sync_copy(acc.at[pl.ds(TSTART, TAIL)], bounce.at[pl.ds(0, TAIL)])
    pltpu.sync_copy(bounce.at[pl.ds(0, TAIL)], out_hbm.at[c, pl.ds(TSTART, TAIL)])


def _sc_scatter(m, src3, dst3, attr, zeros):
  mesh = plsc.VectorSubcoreMesh(core_axis_name="c", subcore_axis_name="s")
  k = pl.kernel(
      _sc_body,
      out_type=jax.ShapeDtypeStruct((NC, N, D), jnp.float32),
      mesh=mesh,
      scratch_types=[
          pltpu.VMEM((SUP * CB,), jnp.int32),
          pltpu.VMEM((NBUF, CB), jnp.int32),
          pltpu.VMEM((SUP * CB,), jnp.float32),
          pltpu.VMEM((NBUF, CB, D), jnp.float32),
          pltpu.VMEM((CHW, D), jnp.float32),
          pltpu.VMEM_SHARED((N, D), jnp.float32),
          pltpu.SemaphoreType.DMA((NBUF,)),
          pltpu.SemaphoreType.DMA((NBUF,)),
          pltpu.SemaphoreType.DMA((NBUF,)),
      ],
      compiler_params=pltpu.CompilerParams(needs_layout_passes=False),
  )
  return k(m, src3, dst3, attr, zeros)


BN = 2000


def _mm_body(h_ref, w_ref, o_ref):
  o_ref[...] = jnp.dot(h_ref[...], w_ref[...],
                       preferred_element_type=jnp.float32)


def _matmul(h, w):
  return pl.pallas_call(
      _mm_body,
      grid=(N // BN,),
      in_specs=[
          pl.BlockSpec((BN, D), lambda i: (i, 0)),
          pl.BlockSpec((D, D), lambda i: (0, 0)),
      ],
      out_specs=pl.BlockSpec((BN, D), lambda i: (i, 0)),
      out_shape=jax.ShapeDtypeStruct((N, D), jnp.float32),
  )(h, w)


def _gru_math(p_ref, h_ref, wih_ref, whh_ref, bih_ref, bhh_ref):
  agg = p_ref[0] + p_ref[1]
  h = h_ref[...]
  gi = lax.dot_general(agg, wih_ref[...], (((1,), (1,)), ((), ())),
                       preferred_element_type=jnp.float32) + bih_ref[...]
  gh = lax.dot_general(h, whh_ref[...], (((1,), (1,)), ((), ())),
                       preferred_element_type=jnp.float32) + bhh_ref[...]
  r = jax.nn.sigmoid(gi[:, :D] + gh[:, :D])
  z = jax.nn.sigmoid(gi[:, D:2 * D] + gh[:, D:2 * D])
  n = jnp.tanh(gi[:, 2 * D:] + r * gh[:, 2 * D:])
  return (1.0 - z) * n + z * h


def _gru_mid_body(p_ref, h_ref, wih_ref, whh_ref, bih_ref, bhh_ref, wn_ref,
                  hn_ref, mn_ref):
  hn = _gru_math(p_ref, h_ref, wih_ref, whh_ref, bih_ref, bhh_ref)
  hn_ref[...] = hn
  mn_ref[...] = jnp.dot(hn, wn_ref[...], preferred_element_type=jnp.float32)


def _gru_last_body(p_ref, h_ref, wih_ref, whh_ref, bih_ref, bhh_ref,
                   fcw_ref, fcb_ref, out_ref):
  hn = _gru_math(p_ref, h_ref, wih_ref, whh_ref, bih_ref, bhh_ref)
  hr = jnp.maximum(hn, 0.0)
  out_ref[...] = jnp.sum(hr * fcw_ref[...], axis=1, keepdims=True) + fcb_ref[...]


_W_SPECS = [
    pl.BlockSpec((NC, BN, D), lambda i: (0, i, 0)),
    pl.BlockSpec((BN, D), lambda i: (i, 0)),
    pl.BlockSpec((3 * D, D), lambda i: (0, 0)),
    pl.BlockSpec((3 * D, D), lambda i: (0, 0)),
    pl.BlockSpec((1, 3 * D), lambda i: (0, 0)),
    pl.BlockSpec((1, 3 * D), lambda i: (0, 0)),
]


def _gru_mid(p, h, wih, whh, bih, bhh, wn):
  return pl.pallas_call(
      _gru_mid_body,
      grid=(N // BN,),
      in_specs=_W_SPECS + [pl.BlockSpec((D, D), lambda i: (0, 0))],
      out_specs=[
          pl.BlockSpec((BN, D), lambda i: (i, 0)),
          pl.BlockSpec((BN, D), lambda i: (i, 0)),
      ],
      out_shape=[
          jax.ShapeDtypeStruct((N, D), jnp.float32),
          jax.ShapeDtypeStruct((N, D), jnp.float32),
      ],
  )(p, h, wih, whh, bih, bhh, wn)


def _gru_last(p, h, wih, whh, bih, bhh, fcw, fcb):
  return pl.pallas_call(
      _gru_last_body,
      grid=(N // BN,),
      in_specs=_W_SPECS + [
          pl.BlockSpec((1, D), lambda i: (0, 0)),
          pl.BlockSpec((1, 1), lambda i: (0, 0)),
      ],
      out_specs=pl.BlockSpec((BN, 1), lambda i: (i, 0)),
      out_shape=jax.ShapeDtypeStruct((N, 1), jnp.float32),
  )(p, h, wih, whh, bih, bhh, fcw, fcb)


def kernel(x, edge_index, edge_attr, weight, W_ih, W_hh, b_ih, b_hh,
           fc_w, fc_b, prob=0):
  src3 = edge_index[0]
  dst3 = edge_index[1]
  zeros = jnp.zeros((CHW, D), jnp.float32)
  bih = b_ih.reshape(1, 3 * D)
  bhh = b_hh.reshape(1, 3 * D)
  fcb = fc_b.reshape(1, 1)

  m = _matmul(x, weight[0])
  p = _sc_scatter(m, src3, dst3, edge_attr, zeros)
  h, m = _gru_mid(p, x, W_ih, W_hh, bih, bhh, weight[1])
  p = _sc_scatter(m, src3, dst3, edge_attr, zeros)
  out = _gru_last(p, h, W_ih, W_hh, bih, bhh, fc_w, fcb)
  return out


# speedup vs baseline: 10.9334x; 1.0138x over previous
import functools
import jax
import jax.numpy as jnp
from jax import lax
from jax.experimental import pallas as pl
from jax.experimental.pallas import tpu as pltpu
from jax.experimental.pallas import tpu_sc as plsc

N = 10000
E = 320000
D = 128
NC = 2
NS = 16
NW = NC * NS
EW = E // NW
CB = 40
SUP = 50
NSUP = EW // (SUP * CB)
NBUF = 5
GDEP = 4
S624 = 624
CHW = 48
NCHW = 13
TAIL = N - NS * S624
TSTART = NS * S624


def _sc_body(m_hbm, src_hbm, dst_hbm, attr_hbm, zero_hbm, out_hbm,
             srcv, dstidx, attrv, rows, acc, gsem, ssem, dsem):
  c = lax.axis_index("c")
  s = lax.axis_index("s")
  wid = c * NS + s

  start = s * S624
  pltpu.sync_copy(zero_hbm, acc.at[pl.ds(start, S624)])

  @pl.when(s == NS - 1)
  def _zero_tail():
    pltpu.sync_copy(zero_hbm.at[pl.ds(0, TAIL)], acc.at[pl.ds(TSTART, TAIL)])
  plsc.subcore_barrier()

  def start_gather(sj, g, b):
    pltpu.async_copy(m_hbm.at[srcv.at[pl.ds(g * CB, CB)]], rows.at[b],
                     gsem.at[b])
    pltpu.async_copy(dst_hbm.at[pl.ds(wid * EW + sj * SUP * CB + g * CB, CB)],
                     dstidx.at[b], dsem.at[b])

  def wait_dst(b):
    pltpu.make_async_copy(dst_hbm.at[pl.ds(0, CB)], dstidx.at[b],
                          dsem.at[b]).wait()

  def wait_gather(b):
    pltpu.make_async_copy(m_hbm.at[pl.ds(0, CB)], rows.at[b],
                          gsem.at[b]).wait()

  def wait_scatter(b):
    pltpu.make_async_copy(m_hbm.at[pl.ds(0, CB)], rows.at[b],
                          ssem.at[b]).wait()

  def super_chunk(sj, carry):
    c1 = pltpu.async_copy(
        src_hbm.at[pl.ds(wid * EW + sj * SUP * CB, SUP * CB)], srcv,
        gsem.at[0])
    c2 = pltpu.async_copy(
        attr_hbm.at[pl.ds(wid * EW + sj * SUP * CB, SUP * CB)], attrv,
        gsem.at[1])
    c1.wait()
    c2.wait()
    for b in range(GDEP):
      start_gather(sj, b, b)

    def block(kk, carry2):
      for b in range(NBUF):
        jj = kk * NBUF + b

        wait_gather(b)
        wait_dst(b)

        def edge(e, carry3):
          ab = plsc.load_gather(
              attrv, [jnp.full((16,), jj * CB + e, jnp.int32)])
          for q in range(D // 16):
            rows[b, e, pl.ds(q * 16, 16)] = (
                rows[b, e, pl.ds(q * 16, 16)] * ab)
          return carry3
        lax.fori_loop(0, CB, edge, 0, unroll=2)

        pltpu.async_copy(rows.at[b], acc.at[dstidx.at[b]], ssem.at[b],
                         add=True)

        g = jj + GDEP
        bn = (b + GDEP) % NBUF

        @pl.when((g < SUP) & (jj >= 1))
        def _():
          wait_scatter(bn)

        @pl.when(g < SUP)
        def _():
          start_gather(sj, g, bn)
      return carry2
    lax.fori_loop(0, SUP // NBUF, block, 0)

    for b in range(NBUF):
      wait_scatter(b)
    return carry
  lax.fori_loop(0, NSUP, super_chunk, 0)

  plsc.subcore_barrier()
  pltpu.sync_copy(acc.at[pl.ds(start, S624)],
                  out_hbm.at[c, pl.ds(start, S624)])

  @pl.when(s == NS - 1)
  def _write_tail():
    pltpu.sync_copy(acc.at[pl.ds(TSTART, TAIL)],
                    out_hbm.at[c, pl.ds(TSTART, TAIL)])


def _sc_scatter(m, src3, dst3, attr, zeros):
  mesh = plsc.VectorSubcoreMesh(core_axis_name="c", subcore_axis_name="s")
  k = pl.kernel(
      _sc_body,
      out_type=jax.ShapeDtypeStruct((NC, N, D), jnp.float32),
      mesh=mesh,
      scratch_types=[
          pltpu.VMEM((SUP * CB,), jnp.int32),
          pltpu.VMEM((NBUF, CB), jnp.int32),
          pltpu.VMEM((SUP * CB,), jnp.float32),
          pltpu.VMEM((NBUF, CB, D), jnp.float32),
          pltpu.VMEM_SHARED((N, D), jnp.float32),
          pltpu.SemaphoreType.DMA((NBUF,)),
          pltpu.SemaphoreType.DMA((NBUF,)),
          pltpu.SemaphoreType.DMA((NBUF,)),
      ],
      compiler_params=pltpu.CompilerParams(needs_layout_passes=False),
  )
  return k(m, src3, dst3, attr, zeros)


BN = 2000


def _mm_body(h_ref, w_ref, o_ref):
  o_ref[...] = jnp.dot(h_ref[...], w_ref[...],
                       preferred_element_type=jnp.float32)


def _matmul(h, w):
  return pl.pallas_call(
      _mm_body,
      grid=(N // BN,),
      in_specs=[
          pl.BlockSpec((BN, D), lambda i: (i, 0)),
          pl.BlockSpec((D, D), lambda i: (0, 0)),
      ],
      out_specs=pl.BlockSpec((BN, D), lambda i: (i, 0)),
      out_shape=jax.ShapeDtypeStruct((N, D), jnp.float32),
  )(h, w)


def _gru_math(p_ref, h_ref, wih_ref, whh_ref, bih_ref, bhh_ref):
  agg = p_ref[0] + p_ref[1]
  h = h_ref[...]
  gi = lax.dot_general(agg, wih_ref[...], (((1,), (1,)), ((), ())),
                       preferred_element_type=jnp.float32) + bih_ref[...]
  gh = lax.dot_general(h, whh_ref[...], (((1,), (1,)), ((), ())),
                       preferred_element_type=jnp.float32) + bhh_ref[...]
  r = jax.nn.sigmoid(gi[:, :D] + gh[:, :D])
  z = jax.nn.sigmoid(gi[:, D:2 * D] + gh[:, D:2 * D])
  n = jnp.tanh(gi[:, 2 * D:] + r * gh[:, 2 * D:])
  return (1.0 - z) * n + z * h


def _gru_mid_body(p_ref, h_ref, wih_ref, whh_ref, bih_ref, bhh_ref, wn_ref,
                  hn_ref, mn_ref):
  hn = _gru_math(p_ref, h_ref, wih_ref, whh_ref, bih_ref, bhh_ref)
  hn_ref[...] = hn
  mn_ref[...] = jnp.dot(hn, wn_ref[...], preferred_element_type=jnp.float32)


def _gru_last_body(p_ref, h_ref, wih_ref, whh_ref, bih_ref, bhh_ref,
                   fcw_ref, fcb_ref, out_ref):
  hn = _gru_math(p_ref, h_ref, wih_ref, whh_ref, bih_ref, bhh_ref)
  hr = jnp.maximum(hn, 0.0)
  out_ref[...] = jnp.sum(hr * fcw_ref[...], axis=1, keepdims=True) + fcb_ref[...]


_W_SPECS = [
    pl.BlockSpec((NC, BN, D), lambda i: (0, i, 0)),
    pl.BlockSpec((BN, D), lambda i: (i, 0)),
    pl.BlockSpec((3 * D, D), lambda i: (0, 0)),
    pl.BlockSpec((3 * D, D), lambda i: (0, 0)),
    pl.BlockSpec((1, 3 * D), lambda i: (0, 0)),
    pl.BlockSpec((1, 3 * D), lambda i: (0, 0)),
]


def _gru_mid(p, h, wih, whh, bih, bhh, wn):
  return pl.pallas_call(
      _gru_mid_body,
      grid=(N // BN,),
      in_specs=_W_SPECS + [pl.BlockSpec((D, D), lambda i: (0, 0))],
      out_specs=[
          pl.BlockSpec((BN, D), lambda i: (i, 0)),
          pl.BlockSpec((BN, D), lambda i: (i, 0)),
      ],
      out_shape=[
          jax.ShapeDtypeStruct((N, D), jnp.float32),
          jax.ShapeDtypeStruct((N, D), jnp.float32),
      ],
  )(p, h, wih, whh, bih, bhh, wn)


def _gru_last(p, h, wih, whh, bih, bhh, fcw, fcb):
  return pl.pallas_call(
      _gru_last_body,
      grid=(N // BN,),
      in_specs=_W_SPECS + [
          pl.BlockSpec((1, D), lambda i: (0, 0)),
          pl.BlockSpec((1, 1), lambda i: (0, 0)),
      ],
      out_specs=pl.BlockSpec((BN, 1), lambda i: (i, 0)),
      out_shape=jax.ShapeDtypeStruct((N, 1), jnp.float32),
  )(p, h, wih, whh, bih, bhh, fcw, fcb)


def kernel(x, edge_index, edge_attr, weight, W_ih, W_hh, b_ih, b_hh,
           fc_w, fc_b, prob=0):
  src3 = edge_index[0]
  dst3 = edge_index[1]
  zeros = jnp.zeros((S624, D), jnp.float32)
  bih = b_ih.reshape(1, 3 * D)
  bhh = b_hh.reshape(1, 3 * D)
  fcb = fc_b.reshape(1, 1)

  m = _matmul(x, weight[0])
  p = _sc_scatter(m, src3, dst3, edge_attr, zeros)
  h, m = _gru_mid(p, x, W_ih, W_hh, bih, bhh, weight[1])
  p = _sc_scatter(m, src3, dst3, edge_attr, zeros)
  out = _gru_last(p, h, W_ih, W_hh, bih, bhh, fc_w, fcb)
  return out
